# baseline (speedup 1.0000x reference)
import functools

import jax
import jax.numpy as jnp
from jax import lax
from jax.experimental import pallas as pl
from jax.experimental.pallas import tpu as pltpu
from jax.experimental.pallas import tpu_sc as plsc

DAT_NUM = 1000000
DEL = 64
B = 16384

NC = 2
NS = 16
L = 16
NW = NC * NS
BPW = B // NW
CHUNK = 128
NCH = BPW // CHUNK

_mesh = plsc.VectorSubcoreMesh(core_axis_name="c", subcore_axis_name="s")


@functools.partial(
    pl.kernel,
    mesh=_mesh,
    out_type=jax.ShapeDtypeStruct((B, DEL), jnp.float32),
    scratch_types=[
        pltpu.VMEM((NCH, CHUNK), jnp.int32),
        pltpu.VMEM((BPW, DEL), jnp.float32),
        pltpu.SemaphoreType.DMA,
    ],
    compiler_params=pltpu.CompilerParams(use_tc_tiling_on_sc=False),
)
def _gather_sigmoid(idx_hbm, table_hbm, out_hbm, idx_v, rows_v, sem):
    wid = lax.axis_index("s") * NC + lax.axis_index("c")
    base = wid * BPW

    pltpu.sync_copy(idx_hbm.at[wid], idx_v)

    copies = []
    for j in range(NCH):
        copies.append(
            pltpu.async_copy(
                table_hbm.at[idx_v.at[j]],
                rows_v.at[pl.ds(j * CHUNK, CHUNK)],
                sem,
            )
        )
    for c in copies:
        c.wait()

    def body(i, carry):
        for j in range(DEL // L):
            x = rows_v[i, pl.ds(j * L, L)]
            rows_v[i, pl.ds(j * L, L)] = 1.0 / (1.0 + jnp.exp(-x))
        return carry

    lax.fori_loop(0, BPW, body, 0)

    pltpu.sync_copy(rows_v, out_hbm.at[pl.ds(base, BPW)])


def kernel(idx, embeddings):
    idx3 = idx.astype(jnp.int32).reshape(NW, NCH, CHUNK)
    out = _gather_sigmoid(idx3, embeddings)
    return out[:, :, None]


# speedup vs baseline: 1.5761x; 1.5761x over previous
import functools

import jax
import jax.numpy as jnp
from jax import lax
from jax.experimental import pallas as pl
from jax.experimental.pallas import tpu as pltpu
from jax.experimental.pallas import tpu_sc as plsc

DAT_NUM = 1000000
DEL = 64
B = 16384

NC = 2
NS = 16
L = 16
NW = NC * NS
BPW = B // NW
GRP = 8
NG = BPW // L
RING = 4

_mesh = plsc.VectorSubcoreMesh(core_axis_name="c", subcore_axis_name="s")


@functools.partial(
    pl.kernel,
    mesh=_mesh,
    out_type=jax.ShapeDtypeStruct((B * DEL,), jnp.float32),
    scratch_types=[
        pltpu.VMEM((BPW,), jnp.int32),
        pltpu.VMEM((RING * L, GRP, DEL), jnp.float32),
        pltpu.VMEM((DEL * BPW,), jnp.float32),
        pltpu.SemaphoreType.DMA,
        pltpu.SemaphoreType.DMA,
    ],
    compiler_params=pltpu.CompilerParams(
        use_tc_tiling_on_sc=True, needs_layout_passes=False
    ),
)
def _gather_sigmoid(idx_hbm, table_hbm, drain_hbm, out_hbm, idx_v, blocks_v,
                    outT_v, sem, osem):
    wid = lax.axis_index("s") * NC + lax.axis_index("c")
    base = wid * BPW

    pltpu.sync_copy(idx_hbm.at[pl.ds(base, BPW)], idx_v)

    addr_base = [(lax.iota(jnp.int32, L) + (k * L)) * BPW
                 for k in range(DEL // L)]

    def fire_group(g):
        vec = idx_v[pl.ds(g * L, L)]
        slot0 = lax.rem(g, RING) * L
        for l in range(L):
            gg = vec[l] // GRP
            pltpu.async_copy(
                table_hbm.at[pl.ds(pl.multiple_of(gg * GRP, GRP), GRP)],
                blocks_v.at[slot0 + l],
                sem,
            )

    def process_group(g):
        vec = idx_v[pl.ds(g * L, L)]
        slot0 = lax.rem(g, RING) * L
        pltpu.make_async_copy(
            drain_hbm, blocks_v.at[pl.ds(slot0, L)], sem
        ).wait()
        i0 = g * L
        for l in range(L):
            r = vec[l] % GRP
            for k in range(DEL // L):
                x = blocks_v[slot0 + l, r, pl.ds(k * L, L)]
                y = 1.0 / (1.0 + jnp.exp(-x))
                addr = addr_base[k] + (i0 + l)
                plsc.store_scatter(outT_v, [addr], y)

    for g in range(RING):
        fire_group(g)

    def body(g, carry):
        process_group(g)

        @pl.when(g + RING < NG)
        def _():
            fire_group(g + RING)

        return carry

    lax.fori_loop(0, NG, body, 0)

    for d in range(DEL):
        pltpu.async_copy(
            outT_v.at[pl.ds(d * BPW, BPW)],
            out_hbm.at[pl.ds(d * B + base, BPW)],
            osem,
        )
    for d in range(DEL):
        pltpu.make_async_copy(
            outT_v.at[pl.ds(d * BPW, BPW)],
            out_hbm.at[pl.ds(d * B + base, BPW)],
            osem,
        ).wait()


def kernel(idx, embeddings):
    drain = jnp.zeros((L, GRP, DEL), jnp.float32)
    out_flat = _gather_sigmoid(idx.astype(jnp.int32), embeddings, drain)
    return out_flat.reshape(DEL, 1, B).transpose(2, 0, 1)


# speedup vs baseline: 1.5799x; 1.0025x over previous
import functools

import jax
import jax.numpy as jnp
from jax import lax
from jax.experimental import pallas as pl
from jax.experimental.pallas import tpu as pltpu
from jax.experimental.pallas import tpu_sc as plsc

DAT_NUM = 1000000
DEL = 64
B = 16384

NC = 2
NS = 16
L = 16
NW = NC * NS
BPW = B // NW
GRP = 8
NG = BPW // L
RING = 5

_mesh = plsc.VectorSubcoreMesh(core_axis_name="c", subcore_axis_name="s")


@functools.partial(
    pl.kernel,
    mesh=_mesh,
    out_type=jax.ShapeDtypeStruct((B * DEL,), jnp.float32),
    scratch_types=[
        pltpu.VMEM((BPW,), jnp.int32),
        pltpu.VMEM((RING * L, GRP, DEL), jnp.float32),
        pltpu.VMEM((DEL * BPW,), jnp.float32),
        pltpu.SemaphoreType.DMA,
        pltpu.SemaphoreType.DMA,
    ],
    compiler_params=pltpu.CompilerParams(
        use_tc_tiling_on_sc=True, needs_layout_passes=False
    ),
)
def _gather_sigmoid(idx_hbm, table_hbm, drain_hbm, out_hbm, idx_v, blocks_v,
                    outT_v, sem, osem):
    wid = lax.axis_index("s") * NC + lax.axis_index("c")
    base = wid * BPW

    pltpu.sync_copy(idx_hbm.at[pl.ds(base, BPW)], idx_v)

    addr_base = [(lax.iota(jnp.int32, L) + (k * L)) * BPW
                 for k in range(DEL // L)]

    def fire_group(g):
        vec = idx_v[pl.ds(g * L, L)]
        slot0 = lax.rem(g, RING) * L
        for l in range(L):
            gg = vec[l] // GRP
            pltpu.async_copy(
                table_hbm.at[pl.ds(pl.multiple_of(gg * GRP, GRP), GRP)],
                blocks_v.at[slot0 + l],
                sem,
            )

    def process_group(g):
        vec = idx_v[pl.ds(g * L, L)]
        slot0 = lax.rem(g, RING) * L
        pltpu.make_async_copy(
            drain_hbm, blocks_v.at[pl.ds(slot0, L)], sem
        ).wait()
        i0 = g * L
        for l in range(L):
            r = vec[l] % GRP
            for k in range(DEL // L):
                x = blocks_v[slot0 + l, r, pl.ds(k * L, L)]
                y = 1.0 / (1.0 + jnp.exp(-x))
                addr = addr_base[k] + (i0 + l)
                plsc.store_scatter(outT_v, [addr], y)

    for g in range(RING):
        fire_group(g)

    def body(g, carry):
        process_group(g)

        @pl.when(g + RING < NG)
        def _():
            fire_group(g + RING)

        return carry

    lax.fori_loop(0, NG, body, 0)

    for d in range(DEL):
        pltpu.async_copy(
            outT_v.at[pl.ds(d * BPW, BPW)],
            out_hbm.at[pl.ds(d * B + base, BPW)],
            osem,
        )
    for d in range(DEL):
        pltpu.make_async_copy(
            outT_v.at[pl.ds(d * BPW, BPW)],
            out_hbm.at[pl.ds(d * B + base, BPW)],
            osem,
        ).wait()


def kernel(idx, embeddings):
    drain = jnp.zeros((L, GRP, DEL), jnp.float32)
    out_flat = _gather_sigmoid(idx.astype(jnp.int32), embeddings, drain)
    return out_flat.reshape(DEL, 1, B).transpose(2, 0, 1)
